# Initial kernel scaffold; baseline (speedup 1.0000x reference)
import jax
import jax.numpy as jnp
from jax.experimental import pallas as pl


def kernel(x, edge_index, W1, b1, W2, b2):
    raise NotImplementedError("write your pallas kernel here")


# speedup vs baseline: 10.9375x; 10.9375x over previous
import functools

import jax
import jax.numpy as jnp
from jax import lax
from jax.experimental import pallas as pl
from jax.experimental.pallas import tpu as pltpu
from jax.experimental.pallas import tpu_sc as plsc

N = 10000
E = 320000
D = 128

NW = 32
BLK = 128
NB = 79
EPAD = NW * NB * BLK
NPAD = 10240
RPT = NPAD // 16

_mesh = plsc.VectorSubcoreMesh(core_axis_name="c", subcore_axis_name="s")


@functools.partial(
    pl.kernel,
    out_type=jax.ShapeDtypeStruct((2, NPAD, 16), jnp.float32),
    mesh=_mesh,
    compiler_params=pltpu.CompilerParams(use_tc_tiling_on_sc=False),
    scratch_types=[
        pltpu.VMEM((NB, BLK), jnp.int32),
        pltpu.VMEM((BLK, 16), jnp.float32),
        pltpu.VMEM((BLK, 16), jnp.float32),
        pltpu.VMEM_SHARED((NPAD, 16), jnp.float32),
    ],
)
def _sc_deg(dst_hbm, ones_hbm, zeros_hbm, out_hbm, idx_v, ones_v, zeros_v, acc):
    c = lax.axis_index("c")
    s = lax.axis_index("s")
    wid = s * 2 + c
    pltpu.sync_copy(dst_hbm.at[wid], idx_v)
    pltpu.sync_copy(ones_hbm, ones_v)
    pltpu.sync_copy(zeros_hbm, zeros_v)
    for b in range(RPT // BLK):
        pltpu.sync_copy(zeros_v, acc.at[pl.ds(s * RPT + b * BLK, BLK)])
    plsc.subcore_barrier()

    def body(j, carry):
        pltpu.sync_copy(ones_v, acc.at[idx_v.at[j]], add=True)
        return carry

    lax.fori_loop(0, NB, body, 0)
    plsc.subcore_barrier()
    pltpu.sync_copy(acc.at[pl.ds(s * RPT, RPT)], out_hbm.at[c, pl.ds(s * RPT, RPT)])


DH = D // 2


@functools.partial(
    pl.kernel,
    out_type=jax.ShapeDtypeStruct((2, 2, NPAD, DH), jnp.float32),
    mesh=_mesh,
    compiler_params=pltpu.CompilerParams(use_tc_tiling_on_sc=False),
    scratch_types=[
        pltpu.VMEM((NB, BLK), jnp.int32),
        pltpu.VMEM((NB, BLK), jnp.int32),
        pltpu.VMEM((BLK, DH), jnp.float32),
        pltpu.VMEM((BLK, DH), jnp.float32),
        pltpu.VMEM_SHARED((NPAD, DH), jnp.float32),
        pltpu.SemaphoreType.DMA,
    ],
)
def _sc_scatter(gl_hbm, gr_hbm, src_hbm, dst_hbm, zeros_hbm, out_hbm,
                src_v, dst_v, rows_v, zeros_v, acc, sem):
    c = lax.axis_index("c")
    s = lax.axis_index("s")
    wid = s * 2 + c
    pltpu.sync_copy(src_hbm.at[wid], src_v)
    pltpu.sync_copy(dst_hbm.at[wid], dst_v)
    pltpu.sync_copy(zeros_hbm, zeros_v)
    for half in range(2):
        g_hbm = gl_hbm if half == 0 else gr_hbm
        for b in range(RPT // BLK):
            pltpu.sync_copy(zeros_v, acc.at[pl.ds(s * RPT + b * BLK, BLK)])
        plsc.subcore_barrier()

        def body(j, carry):
            pltpu.async_copy(g_hbm.at[src_v.at[j]], rows_v, sem).wait()
            pltpu.sync_copy(rows_v, acc.at[dst_v.at[j]], add=True)
            return carry

        lax.fori_loop(0, NB, body, 0)
        plsc.subcore_barrier()
        pltpu.sync_copy(acc.at[pl.ds(s * RPT, RPT)],
                        out_hbm.at[c, half, pl.ds(s * RPT, RPT)])


_TC_BLK = 1024
_TC_GRID = NPAD // _TC_BLK


def _rows_spec(width=D):
    return pl.BlockSpec((_TC_BLK, width), lambda i: (i, 0))


def _full_spec(shape):
    return pl.BlockSpec(shape, lambda i: (0, 0))


def _dinv(pa_ref, pb_ref):
    deg = 1.0 + pa_ref[:, :1] + pb_ref[:, :1]
    return lax.rsqrt(deg)


def _tc1_body(x_ref, w_ref, pa_ref, pb_ref, h_ref, gl_ref, gr_ref):
    h = jnp.dot(x_ref[...], w_ref[...], preferred_element_type=jnp.float32)
    dinv = _dinv(pa_ref, pb_ref)
    h_ref[...] = h
    g = h * dinv
    gl_ref[...] = g[:, :DH]
    gr_ref[...] = g[:, DH:]


def _tc2_body(u0l, u1l, u0r, u1r, h1_ref, pa_ref, pb_ref, b1_ref, w2_ref,
              h2_ref, g2l_ref, g2r_ref):
    dinv = _dinv(pa_ref, pb_ref)
    ssum = jnp.concatenate([u0l[...] + u1l[...], u0r[...] + u1r[...]], axis=1)
    a = dinv * ssum + (dinv * dinv) * h1_ref[...] + b1_ref[...]
    o = jnp.maximum(a, 0.0)
    h2 = jnp.dot(o, w2_ref[...], preferred_element_type=jnp.float32)
    h2_ref[...] = h2
    g2 = h2 * dinv
    g2l_ref[...] = g2[:, :DH]
    g2r_ref[...] = g2[:, DH:]


def _tc3_body(u0l, u1l, u0r, u1r, h2_ref, pa_ref, pb_ref, b2_ref, out_ref):
    dinv = _dinv(pa_ref, pb_ref)
    ssum = jnp.concatenate([u0l[...] + u1l[...], u0r[...] + u1r[...]], axis=1)
    out_ref[...] = dinv * ssum + (dinv * dinv) * h2_ref[...] + b2_ref[...]


_half_out = [jax.ShapeDtypeStruct((NPAD, DH), jnp.float32)] * 2

_tc1 = pl.pallas_call(
    _tc1_body,
    grid=(_TC_GRID,),
    in_specs=[_rows_spec(), _full_spec((D, D)), _rows_spec(16), _rows_spec(16)],
    out_specs=[_rows_spec(), _rows_spec(DH), _rows_spec(DH)],
    out_shape=[jax.ShapeDtypeStruct((NPAD, D), jnp.float32)] + _half_out,
)

_tc2 = pl.pallas_call(
    _tc2_body,
    grid=(_TC_GRID,),
    in_specs=[_rows_spec(DH)] * 4 + [_rows_spec(), _rows_spec(16),
              _rows_spec(16), _full_spec((1, D)), _full_spec((D, D))],
    out_specs=[_rows_spec(), _rows_spec(DH), _rows_spec(DH)],
    out_shape=[jax.ShapeDtypeStruct((NPAD, D), jnp.float32)] + _half_out,
)

_tc3 = pl.pallas_call(
    _tc3_body,
    grid=(_TC_GRID,),
    in_specs=[_rows_spec(DH)] * 4 + [_rows_spec(), _rows_spec(16),
              _rows_spec(16), _full_spec((1, D))],
    out_specs=_rows_spec(),
    out_shape=jax.ShapeDtypeStruct((NPAD, D), jnp.float32),
)


def kernel(x, edge_index, W1, b1, W2, b2):
    ei = edge_index.astype(jnp.int32)
    pad = jnp.full((EPAD - E,), N, dtype=jnp.int32)
    src = jnp.concatenate([ei[0], pad]).reshape(NW, NB, BLK)
    dst = jnp.concatenate([ei[1], pad]).reshape(NW, NB, BLK)
    xp = jnp.pad(x, ((0, NPAD - N), (0, 0)))
    zeros_h = jnp.zeros((BLK, DH), jnp.float32)
    ones16 = jnp.ones((BLK, 16), jnp.float32)
    zeros16 = jnp.zeros((BLK, 16), jnp.float32)

    pdeg = _sc_deg(dst, ones16, zeros16)
    pa, pb = pdeg[0], pdeg[1]

    h1, g1l, g1r = _tc1(xp, W1, pa, pb)
    u1 = _sc_scatter(g1l, g1r, src, dst, zeros_h)
    h2, g2l, g2r = _tc2(u1[0, 0], u1[1, 0], u1[0, 1], u1[1, 1],
                        h1, pa, pb, b1.reshape(1, D), W2)
    u2 = _sc_scatter(g2l, g2r, src, dst, zeros_h)
    out = _tc3(u2[0, 0], u2[1, 0], u2[0, 1], u2[1, 1],
               h2, pa, pb, b2.reshape(1, D))
    return out[:N]


# speedup vs baseline: 12.5238x; 1.1450x over previous
import functools

import jax
import jax.numpy as jnp
from jax import lax
from jax.experimental import pallas as pl
from jax.experimental.pallas import tpu as pltpu
from jax.experimental.pallas import tpu_sc as plsc

N = 10000
E = 320000
D = 128

NW = 32
BLK = 128
NB = 79
EPAD = NW * NB * BLK
NPAD = 10240
RPT = NPAD // 16

_mesh = plsc.VectorSubcoreMesh(core_axis_name="c", subcore_axis_name="s")


@functools.partial(
    pl.kernel,
    out_type=jax.ShapeDtypeStruct((2, NPAD, 16), jnp.float32),
    mesh=_mesh,
    compiler_params=pltpu.CompilerParams(use_tc_tiling_on_sc=False),
    scratch_types=[
        pltpu.VMEM((NB, BLK), jnp.int32),
        pltpu.VMEM((BLK, 16), jnp.float32),
        pltpu.VMEM((BLK, 16), jnp.float32),
        pltpu.VMEM_SHARED((NPAD, 16), jnp.float32),
    ],
)
def _sc_deg(dst_hbm, ones_hbm, zeros_hbm, out_hbm, idx_v, ones_v, zeros_v, acc):
    c = lax.axis_index("c")
    s = lax.axis_index("s")
    wid = s * 2 + c
    pltpu.sync_copy(dst_hbm.at[wid], idx_v)
    pltpu.sync_copy(ones_hbm, ones_v)
    pltpu.sync_copy(zeros_hbm, zeros_v)
    for b in range(RPT // BLK):
        pltpu.sync_copy(zeros_v, acc.at[pl.ds(s * RPT + b * BLK, BLK)])
    plsc.subcore_barrier()

    def body(j, carry):
        pltpu.sync_copy(ones_v, acc.at[idx_v.at[j]], add=True)
        return carry

    lax.fori_loop(0, NB, body, 0)
    plsc.subcore_barrier()
    pltpu.sync_copy(acc.at[pl.ds(s * RPT, RPT)], out_hbm.at[c, pl.ds(s * RPT, RPT)])


DH = D // 2


@functools.partial(
    pl.kernel,
    out_type=jax.ShapeDtypeStruct((2, 2, NPAD, DH), jnp.float32),
    mesh=_mesh,
    compiler_params=pltpu.CompilerParams(use_tc_tiling_on_sc=False),
    scratch_types=[
        pltpu.VMEM((NB, BLK), jnp.int32),
        pltpu.VMEM((NB, BLK), jnp.int32),
        pltpu.VMEM((BLK, DH), jnp.float32),
        pltpu.VMEM((BLK, DH), jnp.float32),
        pltpu.VMEM((BLK, DH), jnp.float32),
        pltpu.VMEM_SHARED((NPAD, DH), jnp.float32),
        pltpu.SemaphoreType.DMA,
        pltpu.SemaphoreType.DMA,
    ],
)
def _sc_scatter(gl_hbm, gr_hbm, src_hbm, dst_hbm, zeros_hbm, out_hbm,
                src_v, dst_v, rows_a, rows_b, zeros_v, acc, sem_a, sem_b):
    c = lax.axis_index("c")
    s = lax.axis_index("s")
    wid = s * 2 + c
    pltpu.sync_copy(src_hbm.at[wid], src_v)
    pltpu.sync_copy(dst_hbm.at[wid], dst_v)
    pltpu.sync_copy(zeros_hbm, zeros_v)
    for half in range(2):
        g_hbm = gl_hbm if half == 0 else gr_hbm

        def fire(j, buf, sem):
            pltpu.async_copy(g_hbm.at[src_v.at[j]], buf, sem)

        def drain(j, buf, sem):
            pltpu.make_async_copy(g_hbm.at[src_v.at[j]], buf, sem).wait()

        def scat(j, buf):
            pltpu.sync_copy(buf, acc.at[dst_v.at[j]], add=True)

        for b in range(RPT // BLK):
            pltpu.sync_copy(zeros_v, acc.at[pl.ds(s * RPT + b * BLK, BLK)])
        plsc.subcore_barrier()

        fire(0, rows_a, sem_a)

        def body(i, carry):
            j0 = 2 * i
            j1 = 2 * i + 1
            drain(j0, rows_a, sem_a)
            fire(j1, rows_b, sem_b)
            scat(j0, rows_a)
            drain(j1, rows_b, sem_b)
            fire(j1 + 1, rows_a, sem_a)
            scat(j1, rows_b)
            return carry

        lax.fori_loop(0, (NB - 1) // 2, body, 0)
        drain(NB - 1, rows_a, sem_a)
        scat(NB - 1, rows_a)
        plsc.subcore_barrier()
        pltpu.sync_copy(acc.at[pl.ds(s * RPT, RPT)],
                        out_hbm.at[c, half, pl.ds(s * RPT, RPT)])


_TC_BLK = 1024
_TC_GRID = NPAD // _TC_BLK


def _rows_spec(width=D):
    return pl.BlockSpec((_TC_BLK, width), lambda i: (i, 0))


def _full_spec(shape):
    return pl.BlockSpec(shape, lambda i: (0, 0))


def _dinv(pa_ref, pb_ref):
    deg = 1.0 + pa_ref[:, :1] + pb_ref[:, :1]
    return lax.rsqrt(deg)


def _tc1_body(x_ref, w_ref, pa_ref, pb_ref, h_ref, gl_ref, gr_ref):
    h = jnp.dot(x_ref[...], w_ref[...], preferred_element_type=jnp.float32)
    dinv = _dinv(pa_ref, pb_ref)
    h_ref[...] = h
    g = h * dinv
    gl_ref[...] = g[:, :DH]
    gr_ref[...] = g[:, DH:]


def _tc2_body(u0l, u1l, u0r, u1r, h1_ref, pa_ref, pb_ref, b1_ref, w2_ref,
              h2_ref, g2l_ref, g2r_ref):
    dinv = _dinv(pa_ref, pb_ref)
    ssum = jnp.concatenate([u0l[...] + u1l[...], u0r[...] + u1r[...]], axis=1)
    a = dinv * ssum + (dinv * dinv) * h1_ref[...] + b1_ref[...]
    o = jnp.maximum(a, 0.0)
    h2 = jnp.dot(o, w2_ref[...], preferred_element_type=jnp.float32)
    h2_ref[...] = h2
    g2 = h2 * dinv
    g2l_ref[...] = g2[:, :DH]
    g2r_ref[...] = g2[:, DH:]


def _tc3_body(u0l, u1l, u0r, u1r, h2_ref, pa_ref, pb_ref, b2_ref, out_ref):
    dinv = _dinv(pa_ref, pb_ref)
    ssum = jnp.concatenate([u0l[...] + u1l[...], u0r[...] + u1r[...]], axis=1)
    out_ref[...] = dinv * ssum + (dinv * dinv) * h2_ref[...] + b2_ref[...]


_half_out = [jax.ShapeDtypeStruct((NPAD, DH), jnp.float32)] * 2

_tc1 = pl.pallas_call(
    _tc1_body,
    grid=(_TC_GRID,),
    in_specs=[_rows_spec(), _full_spec((D, D)), _rows_spec(16), _rows_spec(16)],
    out_specs=[_rows_spec(), _rows_spec(DH), _rows_spec(DH)],
    out_shape=[jax.ShapeDtypeStruct((NPAD, D), jnp.float32)] + _half_out,
)

_tc2 = pl.pallas_call(
    _tc2_body,
    grid=(_TC_GRID,),
    in_specs=[_rows_spec(DH)] * 4 + [_rows_spec(), _rows_spec(16),
              _rows_spec(16), _full_spec((1, D)), _full_spec((D, D))],
    out_specs=[_rows_spec(), _rows_spec(DH), _rows_spec(DH)],
    out_shape=[jax.ShapeDtypeStruct((NPAD, D), jnp.float32)] + _half_out,
)

_tc3 = pl.pallas_call(
    _tc3_body,
    grid=(_TC_GRID,),
    in_specs=[_rows_spec(DH)] * 4 + [_rows_spec(), _rows_spec(16),
              _rows_spec(16), _full_spec((1, D))],
    out_specs=_rows_spec(),
    out_shape=jax.ShapeDtypeStruct((NPAD, D), jnp.float32),
)


def kernel(x, edge_index, W1, b1, W2, b2):
    ei = edge_index.astype(jnp.int32)
    pad = jnp.full((EPAD - E,), N, dtype=jnp.int32)
    src = jnp.concatenate([ei[0], pad]).reshape(NW, NB, BLK)
    dst = jnp.concatenate([ei[1], pad]).reshape(NW, NB, BLK)
    xp = jnp.pad(x, ((0, NPAD - N), (0, 0)))
    zeros_h = jnp.zeros((BLK, DH), jnp.float32)
    ones16 = jnp.ones((BLK, 16), jnp.float32)
    zeros16 = jnp.zeros((BLK, 16), jnp.float32)

    pdeg = _sc_deg(dst, ones16, zeros16)
    pa, pb = pdeg[0], pdeg[1]

    h1, g1l, g1r = _tc1(xp, W1, pa, pb)
    u1 = _sc_scatter(g1l, g1r, src, dst, zeros_h)
    h2, g2l, g2r = _tc2(u1[0, 0], u1[1, 0], u1[0, 1], u1[1, 1],
                        h1, pa, pb, b1.reshape(1, D), W2)
    u2 = _sc_scatter(g2l, g2r, src, dst, zeros_h)
    out = _tc3(u2[0, 0], u2[1, 0], u2[0, 1], u2[1, 1],
               h2, pa, pb, b2.reshape(1, D))
    return out[:N]
